# baseline (speedup 1.0000x reference)
import jax
import jax.numpy as jnp
from jax import lax
from jax.experimental import pallas as pl
from jax.experimental.pallas import tpu as pltpu

B, N, D = 2, 4096, 1024
H, DH = 16, 64
NQ, NKV = 512, 1024
HD = H * DH
NBLK = 512
NB = N // NBLK
BF = jnp.bfloat16
F32 = jnp.float32


def _cumsum_lanes(x):
    L = x.shape[1]
    s = 1
    while s < L:
        shifted = jnp.concatenate(
            [jnp.zeros((1, s), x.dtype), x[:, : L - s]], axis=1)
        x = x + shifted
        s *= 2
    return x


def _select_topk(logits, k):
    bits = lax.bitcast_convert_type(logits, jnp.int32)
    key = jnp.where(bits < 0, bits ^ jnp.int32(0x7FFFFFFF), bits)

    def body(_, carry):
        lo, hi = carry
        x = lo ^ hi
        mid = (lo & hi) + (x >> 1) + (x & 1)
        cnt = jnp.sum((key >= mid).astype(jnp.int32))
        ok = cnt >= k
        return jnp.where(ok, mid, lo), jnp.where(ok, hi, mid - 1)

    lo, _ = lax.fori_loop(
        0, 33, body, (jnp.int32(-2147483648), jnp.int32(2147483647)))
    tau = lo

    gt = (key > tau)
    tie = (key == tau)
    need = k - jnp.sum(gt.astype(jnp.int32))
    tie_cum = _cumsum_lanes(tie.astype(F32))
    sel_b = gt | (tie & (tie_cum <= need.astype(F32)))
    sel = sel_b.astype(F32)
    rank = (_cumsum_lanes(sel) - 1.0).astype(jnp.int32)
    return sel, rank


def _router_kernel(x_ref, wq_ref, wkv_ref,
                   qsel_ref, qrank_ref, qsig_ref,
                   kvsel_ref, kvrank_ref, kvsig_ref):
    x = x_ref[0]
    dn = (((1,), (1,)), ((), ()))
    ql = lax.dot_general(wq_ref[...], x, dn, preferred_element_type=F32)
    kl = lax.dot_general(wkv_ref[...], x, dn, preferred_element_type=F32)
    qsel, qrank = _select_topk(ql, NQ)
    kvsel, kvrank = _select_topk(kl, NKV)
    qsel_ref[0] = qsel
    qrank_ref[0] = qrank
    qsig_ref[0] = jax.nn.sigmoid(ql)
    kvsel_ref[0] = kvsel
    kvrank_ref[0] = kvrank
    kvsig_ref[0] = jax.nn.sigmoid(kl)


def _onehot(rank, sel, rows):
    i = lax.broadcasted_iota(jnp.int32, (rows, rank.shape[1]), 0)
    return jnp.where((i == rank) & (sel > 0.5), 1.0, 0.0).astype(F32)


def _hilo(t):
    hi = t.astype(BF)
    lo = (t - hi.astype(F32)).astype(BF)
    return hi, lo


_DN_NT = (((1,), (1,)), ((), ()))


def _dot3(a, b, dn=None):
    if dn is None:
        dn = (((1,), (0,)), ((), ()))
    ah, al = _hilo(a)
    bh, bl = _hilo(b)
    return (lax.dot_general(ah, bh, dn, preferred_element_type=F32)
            + lax.dot_general(ah, bl, dn, preferred_element_type=F32)
            + lax.dot_general(al, bh, dn, preferred_element_type=F32))


def _dot_exactb(a_bf, b):
    dn = (((1,), (0,)), ((), ()))
    bh, bl = _hilo(b)
    return (lax.dot_general(a_bf, bh, dn, preferred_element_type=F32)
            + lax.dot_general(a_bf, bl, dn, preferred_element_type=F32))


def _gather_kernel(x_ref, qrank_ref, qsel_ref, qsig_ref,
                   kvrank_ref, kvsel_ref, kvsig_ref, rot_ref,
                   qg_ref, qremb_ref, qs_ref,
                   kvg_ref, kvremb_ref, ks_ref):
    xb = x_ref[0]
    xh, xl = _hilo(xb)
    rot = rot_ref[...]
    dn_s = (((1,), (1,)), ((), ()))

    def side(rank_ref, sel_ref, sig_ref, rows):
        oh = _onehot(rank_ref[0, 0:1], sel_ref[0, 0:1], rows)
        ohb = oh.astype(BF)
        g = jnp.dot(ohb, xh, preferred_element_type=F32) \
            + jnp.dot(ohb, xl, preferred_element_type=F32)
        r = jnp.dot(oh, rot, preferred_element_type=F32)
        s = lax.dot_general(oh, sig_ref[0, 0:1], dn_s,
                            preferred_element_type=F32)
        return g, r, s

    qg, qr, qs = side(qrank_ref, qsel_ref, qsig_ref, NQ)
    kg, kr, ks = side(kvrank_ref, kvsel_ref, kvsig_ref, NKV)

    @pl.when(pl.program_id(1) == 0)
    def _init():
        qg_ref[0] = qg
        qremb_ref[0] = qr
        qs_ref[0] = qs
        kvg_ref[0] = kg
        kvremb_ref[0] = kr
        ks_ref[0] = ks

    @pl.when(pl.program_id(1) != 0)
    def _acc():
        qg_ref[0] += qg
        qremb_ref[0] += qr
        qs_ref[0] += qs
        kvg_ref[0] += kg
        kvremb_ref[0] += kr
        ks_ref[0] += ks


def _layernorm(t, gamma):
    mu = jnp.mean(t, axis=1, keepdims=True)
    var = jnp.mean((t - mu) * (t - mu), axis=1, keepdims=True)
    return (t - mu) / jnp.sqrt(var + 1e-5) * gamma


def _rot_mats():
    r = lax.broadcasted_iota(jnp.int32, (HD, HD), 0)
    c = lax.broadcasted_iota(jnp.int32, (HD, HD), 1)
    cm = lax.rem(c, DH)
    p = jnp.where((r == c - DH // 2) & (cm >= DH // 2), 1.0, 0.0) \
        + jnp.where((r == c + DH // 2) & (cm < DH // 2), -1.0, 0.0)
    d = lax.broadcasted_iota(jnp.int32, (DH, HD), 0)
    cc = lax.broadcasted_iota(jnp.int32, (DH, HD), 1)
    tile = jnp.where(lax.rem(cc, DH) == d, 1.0, 0.0)
    return p.astype(F32), tile.astype(F32)


def _apply_rotary(t, remb):
    p, tile = _rot_mats()
    pb = p.astype(BF)
    tileb = tile.astype(BF)
    ch, cl = _hilo(jnp.cos(remb))
    sh, sl = _hilo(jnp.sin(remb))
    dn = (((1,), (0,)), ((), ()))
    ct = lax.dot_general(ch, tileb, dn, preferred_element_type=F32) \
        + lax.dot_general(cl, tileb, dn, preferred_element_type=F32)
    st = lax.dot_general(sh, tileb, dn, preferred_element_type=F32) \
        + lax.dot_general(sl, tileb, dn, preferred_element_type=F32)
    th, tl = _hilo(t)
    rh = lax.dot_general(th, pb, dn, preferred_element_type=F32) \
        + lax.dot_general(tl, pb, dn, preferred_element_type=F32)
    return t * ct + rh * st


def _proj_rot_kernel(g_ref, remb_ref, gamma_ref, w_ref, out_ref):
    tn = _layernorm(g_ref[0], gamma_ref[...])
    t = _dot3(tn, w_ref[...])
    out_ref[0] = _apply_rotary(t, remb_ref[0])


def _proj_v_kernel(g_ref, sc_ref, gamma_ref, w_ref, out_ref):
    tn = _layernorm(g_ref[0], gamma_ref[...])
    t = _dot3(tn, w_ref[...])
    out_ref[0] = t * sc_ref[0]


def _attn_kernel(q_ref, k_ref, v_ref, o_ref):
    outs = []
    for i in range(2):
        q = q_ref[0][:, i * DH:(i + 1) * DH]
        k = k_ref[0][:, i * DH:(i + 1) * DH]
        v = v_ref[0][:, i * DH:(i + 1) * DH]
        sim = _dot3(q, k, _DN_NT)
        sim = sim * (DH ** -0.5)
        m = jnp.max(sim, axis=1, keepdims=True)
        e = jnp.exp(sim - m)
        a = e / jnp.sum(e, axis=1, keepdims=True)
        outs.append(_dot3(a, v))
    o_ref[0] = jnp.concatenate(outs, axis=1)


def _out_kernel(ao_ref, wo_ref, qs_ref, rank_ref, sel_ref, null_ref,
                out_ref, oh_scr, ol_scr):
    @pl.when(pl.program_id(1) == 0)
    def _project():
        o = _dot3(ao_ref[0], wo_ref[...])
        o = o * qs_ref[0]
        hi, lo = _hilo(o)
        oh_scr[...] = hi
        ol_scr[...] = lo

    oh = _onehot(rank_ref[0, 0:1], sel_ref[0, 0:1], NQ)
    ohb = oh.astype(BF)
    dn_t = (((0,), (0,)), ((), ()))
    scat = lax.dot_general(ohb, oh_scr[...], dn_t,
                           preferred_element_type=F32) \
        + lax.dot_general(ohb, ol_scr[...], dn_t,
                          preferred_element_type=F32)
    selc = lax.dot_general(oh, jnp.ones((NQ, 1), F32), dn_t,
                           preferred_element_type=F32)
    out_ref[0] = scat + (1.0 - selc) * null_ref[...]


def _row_spec():
    return pl.BlockSpec((1, 1, N), lambda b: (b, 0, 0))


def _row_blk_spec():
    return pl.BlockSpec((1, 1, NBLK), lambda b, nb: (b, 0, nb))


def _full_spec(shape):
    return pl.BlockSpec(shape, lambda *_: (0,) * len(shape))


def _proj_rot(g, remb, g2, w, rows):
    return pl.pallas_call(
        _proj_rot_kernel,
        grid=(B,),
        in_specs=[
            pl.BlockSpec((1, rows, D), lambda b: (b, 0, 0)),
            pl.BlockSpec((1, rows, DH), lambda b: (b, 0, 0)),
            _full_spec((1, D)),
            _full_spec((D, HD)),
        ],
        out_specs=pl.BlockSpec((1, rows, HD), lambda b: (b, 0, 0)),
        out_shape=jax.ShapeDtypeStruct((B, rows, HD), F32),
    )(g, remb, g2, w)


@jax.jit
def kernel(x, rotary_emb, w_q_router, w_kv_router, ln_gamma, Wq, Wk, Wv, Wo,
           null_tokens):
    wq2 = w_q_router.reshape(1, D)
    wkv2 = w_kv_router.reshape(1, D)
    g2 = ln_gamma.reshape(1, D)
    null2 = null_tokens.reshape(1, D)

    row_f = jax.ShapeDtypeStruct((B, 1, N), F32)
    row_i = jax.ShapeDtypeStruct((B, 1, N), jnp.int32)
    qsel, qrank, qsig, kvsel, kvrank, kvsig = pl.pallas_call(
        _router_kernel,
        grid=(B,),
        in_specs=[
            pl.BlockSpec((1, N, D), lambda b: (b, 0, 0)),
            _full_spec((1, D)),
            _full_spec((1, D)),
        ],
        out_specs=[_row_spec()] * 6,
        out_shape=[row_f, row_i, row_f, row_f, row_i, row_f],
    )(x, wq2, wkv2)

    qg, qremb, qs, kvg, kvremb, ks = pl.pallas_call(
        _gather_kernel,
        grid=(B, NB),
        in_specs=[
            pl.BlockSpec((1, NBLK, D), lambda b, nb: (b, nb, 0)),
            _row_blk_spec(), _row_blk_spec(), _row_blk_spec(),
            _row_blk_spec(), _row_blk_spec(), _row_blk_spec(),
            pl.BlockSpec((NBLK, DH), lambda b, nb: (nb, 0)),
        ],
        out_specs=[
            pl.BlockSpec((1, NQ, D), lambda b, nb: (b, 0, 0)),
            pl.BlockSpec((1, NQ, DH), lambda b, nb: (b, 0, 0)),
            pl.BlockSpec((1, NQ, 1), lambda b, nb: (b, 0, 0)),
            pl.BlockSpec((1, NKV, D), lambda b, nb: (b, 0, 0)),
            pl.BlockSpec((1, NKV, DH), lambda b, nb: (b, 0, 0)),
            pl.BlockSpec((1, NKV, 1), lambda b, nb: (b, 0, 0)),
        ],
        out_shape=[
            jax.ShapeDtypeStruct((B, NQ, D), F32),
            jax.ShapeDtypeStruct((B, NQ, DH), F32),
            jax.ShapeDtypeStruct((B, NQ, 1), F32),
            jax.ShapeDtypeStruct((B, NKV, D), F32),
            jax.ShapeDtypeStruct((B, NKV, DH), F32),
            jax.ShapeDtypeStruct((B, NKV, 1), F32),
        ],
    )(x, qrank, qsel, qsig, kvrank, kvsel, kvsig, rotary_emb)

    q = _proj_rot(qg, qremb, g2, Wq, NQ)
    k = _proj_rot(kvg, kvremb, g2, Wk, NKV)
    v = pl.pallas_call(
        _proj_v_kernel,
        grid=(B,),
        in_specs=[
            pl.BlockSpec((1, NKV, D), lambda b: (b, 0, 0)),
            pl.BlockSpec((1, NKV, 1), lambda b: (b, 0, 0)),
            _full_spec((1, D)),
            _full_spec((D, HD)),
        ],
        out_specs=pl.BlockSpec((1, NKV, HD), lambda b: (b, 0, 0)),
        out_shape=jax.ShapeDtypeStruct((B, NKV, HD), F32),
    )(kvg, ks, g2, Wv)

    ao = pl.pallas_call(
        _attn_kernel,
        grid=(B, H // 2),
        in_specs=[
            pl.BlockSpec((1, NQ, 2 * DH), lambda b, h: (b, 0, h)),
            pl.BlockSpec((1, NKV, 2 * DH), lambda b, h: (b, 0, h)),
            pl.BlockSpec((1, NKV, 2 * DH), lambda b, h: (b, 0, h)),
        ],
        out_specs=pl.BlockSpec((1, NQ, 2 * DH), lambda b, h: (b, 0, h)),
        out_shape=jax.ShapeDtypeStruct((B, NQ, HD), F32),
    )(q, k, v)

    result = pl.pallas_call(
        _out_kernel,
        grid=(B, NB),
        in_specs=[
            pl.BlockSpec((1, NQ, HD), lambda b, nb: (b, 0, 0)),
            _full_spec((HD, D)),
            pl.BlockSpec((1, NQ, 1), lambda b, nb: (b, 0, 0)),
            _row_blk_spec(), _row_blk_spec(),
            _full_spec((1, D)),
        ],
        out_specs=pl.BlockSpec((1, NBLK, D), lambda b, nb: (b, nb, 0)),
        out_shape=jax.ShapeDtypeStruct((B, N, D), F32),
        scratch_shapes=[pltpu.VMEM((NQ, D), BF), pltpu.VMEM((NQ, D), BF)],
    )(ao, Wo, qs, qrank, qsel, null2)

    return result


# speedup vs baseline: 1.4829x; 1.4829x over previous
import jax
import jax.numpy as jnp
from jax import lax
from jax.experimental import pallas as pl
from jax.experimental.pallas import tpu as pltpu

B, N, D = 2, 4096, 1024
H, DH = 16, 64
NQ, NKV = 512, 1024
HD = H * DH
NBLK = 512
NB = N // NBLK
BF = jnp.bfloat16
F32 = jnp.float32


def _cumsum_lanes(x):
    R, L = x.shape
    s = 1
    while s < L:
        shifted = jnp.concatenate(
            [jnp.zeros((R, s), x.dtype), x[:, : L - s]], axis=1)
        x = x + shifted
        s *= 2
    return x


def _select_topk_rows(logits, kvec):
    bits = lax.bitcast_convert_type(logits, jnp.int32)
    key = jnp.where(bits < 0, bits ^ jnp.int32(0x7FFFFFFF), bits)
    R = key.shape[0]

    def body(_, carry):
        lo, hi = carry
        x = lo ^ hi
        mid = (lo & hi) + (x >> 1) + (x & 1)
        cnt = jnp.sum((key >= mid).astype(jnp.int32), axis=1, keepdims=True)
        ok = cnt >= kvec
        return jnp.where(ok, mid, lo), jnp.where(ok, hi, mid - 1)

    init = (jnp.full((R, 1), -2147483648, jnp.int32),
            jnp.full((R, 1), 2147483647, jnp.int32))
    lo, _ = lax.fori_loop(0, 33, body, init)
    tau = lo

    gt = (key > tau)
    tie = (key == tau)
    need = kvec - jnp.sum(gt.astype(jnp.int32), axis=1, keepdims=True)
    tie_cum = _cumsum_lanes(tie.astype(F32))
    sel_b = gt | (tie & (tie_cum <= need.astype(F32)))
    sel = sel_b.astype(F32)
    rank = (_cumsum_lanes(sel) - 1.0).astype(jnp.int32)
    return sel, rank


def _router_kernel(x_ref, wq_ref, wkv_ref,
                   qsel_ref, qrank_ref, qsig_ref,
                   kvsel_ref, kvrank_ref, kvsig_ref):
    dn = (((1,), (1,)), ((), ()))
    rows = []
    for w_ref in (wq_ref, wkv_ref):
        for b in range(B):
            rows.append(lax.dot_general(w_ref[...], x_ref[b], dn,
                                        preferred_element_type=F32))
    logits = jnp.concatenate(rows, axis=0)
    kvec = jnp.where(
        lax.broadcasted_iota(jnp.int32, (2 * B, 1), 0) < B, NQ, NKV)
    sel, rank = _select_topk_rows(logits, kvec)
    sig = jax.nn.sigmoid(logits)
    for b in range(B):
        qsel_ref[b] = sel[b:b + 1]
        qrank_ref[b] = rank[b:b + 1]
        qsig_ref[b] = sig[b:b + 1]
        kvsel_ref[b] = sel[B + b:B + b + 1]
        kvrank_ref[b] = rank[B + b:B + b + 1]
        kvsig_ref[b] = sig[B + b:B + b + 1]


def _onehot(rank, sel, rows):
    i = lax.broadcasted_iota(jnp.int32, (rows, rank.shape[1]), 0)
    return jnp.where((i == rank) & (sel > 0.5), 1.0, 0.0).astype(F32)


def _hilo(t):
    hi = t.astype(BF)
    lo = (t - hi.astype(F32)).astype(BF)
    return hi, lo


def _gather_kernel(x_ref, qrank_ref, qsel_ref, qsig_ref,
                   kvrank_ref, kvsel_ref, kvsig_ref, rot_ref,
                   qg_ref, qremb_ref, qs_ref,
                   kvg_ref, kvremb_ref, ks_ref):
    xb = x_ref[0]
    xh, xl = _hilo(xb)
    rot = rot_ref[...]
    dn_s = (((1,), (1,)), ((), ()))

    def side(rank_ref, sel_ref, sig_ref, rows):
        oh = _onehot(rank_ref[0, 0:1], sel_ref[0, 0:1], rows)
        ohb = oh.astype(BF)
        g = jnp.dot(ohb, xh, preferred_element_type=F32) \
            + jnp.dot(ohb, xl, preferred_element_type=F32)
        r = jnp.dot(oh, rot, preferred_element_type=F32)
        s = lax.dot_general(oh, sig_ref[0, 0:1], dn_s,
                            preferred_element_type=F32)
        return g, r, s

    qg, qr, qs = side(qrank_ref, qsel_ref, qsig_ref, NQ)
    kg, kr, ks = side(kvrank_ref, kvsel_ref, kvsig_ref, NKV)

    @pl.when(pl.program_id(1) == 0)
    def _init():
        qg_ref[0] = qg
        qremb_ref[0] = qr
        qs_ref[0] = qs
        kvg_ref[0] = kg
        kvremb_ref[0] = kr
        ks_ref[0] = ks

    @pl.when(pl.program_id(1) != 0)
    def _acc():
        qg_ref[0] += qg
        qremb_ref[0] += qr
        qs_ref[0] += qs
        kvg_ref[0] += kg
        kvremb_ref[0] += kr
        ks_ref[0] += ks


def _layernorm(t, gamma):
    mu = jnp.mean(t, axis=1, keepdims=True)
    var = jnp.mean((t - mu) * (t - mu), axis=1, keepdims=True)
    return (t - mu) / jnp.sqrt(var + 1e-5) * gamma


def _rot_mats():
    r = lax.broadcasted_iota(jnp.int32, (HD, HD), 0)
    c = lax.broadcasted_iota(jnp.int32, (HD, HD), 1)
    cm = lax.rem(c, DH)
    p = jnp.where((r == c - DH // 2) & (cm >= DH // 2), 1.0, 0.0) \
        + jnp.where((r == c + DH // 2) & (cm < DH // 2), -1.0, 0.0)
    d = lax.broadcasted_iota(jnp.int32, (DH, HD), 0)
    cc = lax.broadcasted_iota(jnp.int32, (DH, HD), 1)
    tile = jnp.where(lax.rem(cc, DH) == d, 1.0, 0.0)
    return p.astype(F32), tile.astype(F32)


def _apply_rotary(t, remb):
    p, tile = _rot_mats()
    ct = jnp.dot(jnp.cos(remb), tile, preferred_element_type=F32)
    st = jnp.dot(jnp.sin(remb), tile, preferred_element_type=F32)
    rh = jnp.dot(t, p, preferred_element_type=F32)
    return t * ct + rh * st


def _proj_rot_kernel(g_ref, remb_ref, gamma_ref, w_ref, out_ref):
    tn = _layernorm(g_ref[0], gamma_ref[...])
    t = jnp.dot(tn, w_ref[...], preferred_element_type=F32)
    out_ref[0] = _apply_rotary(t, remb_ref[0])


def _proj_v_kernel(g_ref, sc_ref, gamma_ref, w_ref, out_ref):
    tn = _layernorm(g_ref[0], gamma_ref[...])
    t = jnp.dot(tn, w_ref[...], preferred_element_type=F32)
    out_ref[0] = t * sc_ref[0]


def _attn_kernel(q_ref, k_ref, v_ref, o_ref):
    outs = []
    for i in range(2):
        q = q_ref[0][:, i * DH:(i + 1) * DH]
        k = k_ref[0][:, i * DH:(i + 1) * DH]
        v = v_ref[0][:, i * DH:(i + 1) * DH]
        sim = lax.dot_general(q, k, (((1,), (1,)), ((), ())),
                              preferred_element_type=F32)
        sim = sim * (DH ** -0.5)
        m = jnp.max(sim, axis=1, keepdims=True)
        e = jnp.exp(sim - m)
        a = e / jnp.sum(e, axis=1, keepdims=True)
        outs.append(jnp.dot(a, v, preferred_element_type=F32))
    o_ref[0] = jnp.concatenate(outs, axis=1)


def _out_kernel(ao_ref, wo_ref, qs_ref, rank_ref, sel_ref, null_ref,
                out_ref, oh_scr, ol_scr):
    @pl.when(pl.program_id(1) == 0)
    def _project():
        o = jnp.dot(ao_ref[0], wo_ref[...], preferred_element_type=F32)
        o = o * qs_ref[0]
        hi, lo = _hilo(o)
        oh_scr[...] = hi
        ol_scr[...] = lo

    oh = _onehot(rank_ref[0, 0:1], sel_ref[0, 0:1], NQ)
    ohb = oh.astype(BF)
    dn_t = (((0,), (0,)), ((), ()))
    scat = lax.dot_general(ohb, oh_scr[...], dn_t,
                           preferred_element_type=F32) \
        + lax.dot_general(ohb, ol_scr[...], dn_t,
                          preferred_element_type=F32)
    selc = lax.dot_general(oh, jnp.ones((NQ, 1), F32), dn_t,
                           preferred_element_type=F32)
    out_ref[0] = scat + (1.0 - selc) * null_ref[...]


def _row_spec():
    return pl.BlockSpec((1, 1, N), lambda b: (b, 0, 0))


def _row_blk_spec():
    return pl.BlockSpec((1, 1, NBLK), lambda b, nb: (b, 0, nb))


def _full_spec(shape):
    return pl.BlockSpec(shape, lambda *_: (0,) * len(shape))


def _proj_rot(g, remb, g2, w, rows):
    return pl.pallas_call(
        _proj_rot_kernel,
        grid=(B,),
        in_specs=[
            pl.BlockSpec((1, rows, D), lambda b: (b, 0, 0)),
            pl.BlockSpec((1, rows, DH), lambda b: (b, 0, 0)),
            _full_spec((1, D)),
            _full_spec((D, HD)),
        ],
        out_specs=pl.BlockSpec((1, rows, HD), lambda b: (b, 0, 0)),
        out_shape=jax.ShapeDtypeStruct((B, rows, HD), F32),
    )(g, remb, g2, w)


@jax.jit
def kernel(x, rotary_emb, w_q_router, w_kv_router, ln_gamma, Wq, Wk, Wv, Wo,
           null_tokens):
    wq2 = w_q_router.reshape(1, D)
    wkv2 = w_kv_router.reshape(1, D)
    g2 = ln_gamma.reshape(1, D)
    null2 = null_tokens.reshape(1, D)

    row_f = jax.ShapeDtypeStruct((B, 1, N), F32)
    row_i = jax.ShapeDtypeStruct((B, 1, N), jnp.int32)
    qsel, qrank, qsig, kvsel, kvrank, kvsig = pl.pallas_call(
        _router_kernel,
        grid=(1,),
        in_specs=[
            _full_spec((B, N, D)),
            _full_spec((1, D)),
            _full_spec((1, D)),
        ],
        out_specs=[_full_spec((B, 1, N))] * 6,
        out_shape=[row_f, row_i, row_f, row_f, row_i, row_f],
    )(x, wq2, wkv2)

    qg, qremb, qs, kvg, kvremb, ks = pl.pallas_call(
        _gather_kernel,
        grid=(B, NB),
        in_specs=[
            pl.BlockSpec((1, NBLK, D), lambda b, nb: (b, nb, 0)),
            _row_blk_spec(), _row_blk_spec(), _row_blk_spec(),
            _row_blk_spec(), _row_blk_spec(), _row_blk_spec(),
            pl.BlockSpec((NBLK, DH), lambda b, nb: (nb, 0)),
        ],
        out_specs=[
            pl.BlockSpec((1, NQ, D), lambda b, nb: (b, 0, 0)),
            pl.BlockSpec((1, NQ, DH), lambda b, nb: (b, 0, 0)),
            pl.BlockSpec((1, NQ, 1), lambda b, nb: (b, 0, 0)),
            pl.BlockSpec((1, NKV, D), lambda b, nb: (b, 0, 0)),
            pl.BlockSpec((1, NKV, DH), lambda b, nb: (b, 0, 0)),
            pl.BlockSpec((1, NKV, 1), lambda b, nb: (b, 0, 0)),
        ],
        out_shape=[
            jax.ShapeDtypeStruct((B, NQ, D), F32),
            jax.ShapeDtypeStruct((B, NQ, DH), F32),
            jax.ShapeDtypeStruct((B, NQ, 1), F32),
            jax.ShapeDtypeStruct((B, NKV, D), F32),
            jax.ShapeDtypeStruct((B, NKV, DH), F32),
            jax.ShapeDtypeStruct((B, NKV, 1), F32),
        ],
    )(x, qrank, qsel, qsig, kvrank, kvsel, kvsig, rotary_emb)

    q = _proj_rot(qg, qremb, g2, Wq, NQ)
    k = _proj_rot(kvg, kvremb, g2, Wk, NKV)
    v = pl.pallas_call(
        _proj_v_kernel,
        grid=(B,),
        in_specs=[
            pl.BlockSpec((1, NKV, D), lambda b: (b, 0, 0)),
            pl.BlockSpec((1, NKV, 1), lambda b: (b, 0, 0)),
            _full_spec((1, D)),
            _full_spec((D, HD)),
        ],
        out_specs=pl.BlockSpec((1, NKV, HD), lambda b: (b, 0, 0)),
        out_shape=jax.ShapeDtypeStruct((B, NKV, HD), F32),
    )(kvg, ks, g2, Wv)

    ao = pl.pallas_call(
        _attn_kernel,
        grid=(B, H // 2),
        in_specs=[
            pl.BlockSpec((1, NQ, 2 * DH), lambda b, h: (b, 0, h)),
            pl.BlockSpec((1, NKV, 2 * DH), lambda b, h: (b, 0, h)),
            pl.BlockSpec((1, NKV, 2 * DH), lambda b, h: (b, 0, h)),
        ],
        out_specs=pl.BlockSpec((1, NQ, 2 * DH), lambda b, h: (b, 0, h)),
        out_shape=jax.ShapeDtypeStruct((B, NQ, HD), F32),
    )(q, k, v)

    result = pl.pallas_call(
        _out_kernel,
        grid=(B, NB),
        in_specs=[
            pl.BlockSpec((1, NQ, HD), lambda b, nb: (b, 0, 0)),
            _full_spec((HD, D)),
            pl.BlockSpec((1, NQ, 1), lambda b, nb: (b, 0, 0)),
            _row_blk_spec(), _row_blk_spec(),
            _full_spec((1, D)),
        ],
        out_specs=pl.BlockSpec((1, NBLK, D), lambda b, nb: (b, nb, 0)),
        out_shape=jax.ShapeDtypeStruct((B, N, D), F32),
        scratch_shapes=[pltpu.VMEM((NQ, D), BF), pltpu.VMEM((NQ, D), BF)],
    )(ao, Wo, qs, qrank, qsel, null2)

    return result


# speedup vs baseline: 1.5193x; 1.0245x over previous
import jax
import jax.numpy as jnp
from jax import lax
from jax.experimental import pallas as pl
from jax.experimental.pallas import tpu as pltpu

B, N, D = 2, 4096, 1024
H, DH = 16, 64
NQ, NKV = 512, 1024
HD = H * DH
NBLK = 512
NB = N // NBLK
BF = jnp.bfloat16
F32 = jnp.float32


def _cumsum_lanes(x):
    R, L = x.shape
    s = 1
    while s < L:
        shifted = jnp.concatenate(
            [jnp.zeros((R, s), x.dtype), x[:, : L - s]], axis=1)
        x = x + shifted
        s *= 2
    return x


def _select_topk_rows(logits, kvec):
    bits = lax.bitcast_convert_type(logits, jnp.int32)
    key = jnp.where(bits < 0, bits ^ jnp.int32(0x7FFFFFFF), bits)
    R = key.shape[0]

    def body(_, carry):
        lo, hi = carry
        x = lo ^ hi
        mid = (lo & hi) + (x >> 1) + (x & 1)
        cnt = jnp.sum((key >= mid).astype(jnp.int32), axis=1, keepdims=True)
        ok = cnt >= kvec
        return jnp.where(ok, mid, lo), jnp.where(ok, hi, mid - 1)

    init = (jnp.full((R, 1), -2147483648, jnp.int32),
            jnp.full((R, 1), 2147483647, jnp.int32))
    lo, _ = lax.fori_loop(0, 33, body, init)
    tau = lo

    gt = (key > tau)
    tie = (key == tau)
    need = kvec - jnp.sum(gt.astype(jnp.int32), axis=1, keepdims=True)
    tie_cum = _cumsum_lanes(tie.astype(F32))
    sel_b = gt | (tie & (tie_cum <= need.astype(F32)))
    sel = sel_b.astype(F32)
    rank = (_cumsum_lanes(sel) - 1.0).astype(jnp.int32)
    return sel, rank


def _router_kernel(x_ref, wq_ref, wkv_ref,
                   qsel_ref, qrank_ref, qsig_ref,
                   kvsel_ref, kvrank_ref, kvsig_ref):
    dn = (((1,), (1,)), ((), ()))
    rows = []
    for w_ref in (wq_ref, wkv_ref):
        for b in range(B):
            rows.append(lax.dot_general(w_ref[...], x_ref[b], dn,
                                        preferred_element_type=F32))
    logits = jnp.concatenate(rows, axis=0)
    kvec = jnp.where(
        lax.broadcasted_iota(jnp.int32, (2 * B, 1), 0) < B, NQ, NKV)
    sel, rank = _select_topk_rows(logits, kvec)
    sig = jax.nn.sigmoid(logits)
    for b in range(B):
        qsel_ref[b] = sel[b:b + 1]
        qrank_ref[b] = rank[b:b + 1]
        qsig_ref[b] = sig[b:b + 1]
        kvsel_ref[b] = sel[B + b:B + b + 1]
        kvrank_ref[b] = rank[B + b:B + b + 1]
        kvsig_ref[b] = sig[B + b:B + b + 1]


def _onehot(rank, sel, rows):
    i = lax.broadcasted_iota(jnp.int32, (rows, rank.shape[1]), 0)
    return jnp.where((i == rank) & (sel > 0.5), 1.0, 0.0).astype(F32)


def _hilo(t):
    hi = t.astype(BF)
    lo = (t - hi.astype(F32)).astype(BF)
    return hi, lo


def _gather_kernel(x_ref, qrank_ref, qsel_ref, qsig_ref,
                   kvrank_ref, kvsel_ref, kvsig_ref, rot_ref,
                   qg_ref, qremb_ref, qs_ref,
                   kvg_ref, kvremb_ref, ks_ref):
    xb = x_ref[0]
    xh, xl = _hilo(xb)
    rot = rot_ref[...]
    dn_s = (((1,), (1,)), ((), ()))

    def side(rank_ref, sel_ref, sig_ref, rows):
        oh = _onehot(rank_ref[0, 0:1], sel_ref[0, 0:1], rows)
        ohb = oh.astype(BF)
        g = jnp.dot(ohb, xh, preferred_element_type=F32) \
            + jnp.dot(ohb, xl, preferred_element_type=F32)
        r = jnp.dot(oh, rot, preferred_element_type=F32)
        s = lax.dot_general(oh, sig_ref[0, 0:1], dn_s,
                            preferred_element_type=F32)
        return g, r, s

    qg, qr, qs = side(qrank_ref, qsel_ref, qsig_ref, NQ)
    kg, kr, ks = side(kvrank_ref, kvsel_ref, kvsig_ref, NKV)

    @pl.when(pl.program_id(1) == 0)
    def _init():
        qg_ref[0] = qg
        qremb_ref[0] = qr
        qs_ref[0] = qs
        kvg_ref[0] = kg
        kvremb_ref[0] = kr
        ks_ref[0] = ks

    @pl.when(pl.program_id(1) != 0)
    def _acc():
        qg_ref[0] += qg
        qremb_ref[0] += qr
        qs_ref[0] += qs
        kvg_ref[0] += kg
        kvremb_ref[0] += kr
        ks_ref[0] += ks


def _layernorm(t, gamma):
    mu = jnp.mean(t, axis=1, keepdims=True)
    var = jnp.mean((t - mu) * (t - mu), axis=1, keepdims=True)
    return (t - mu) / jnp.sqrt(var + 1e-5) * gamma


def _rot_mats():
    r = lax.broadcasted_iota(jnp.int32, (HD, HD), 0)
    c = lax.broadcasted_iota(jnp.int32, (HD, HD), 1)
    cm = lax.rem(c, DH)
    p = jnp.where((r == c - DH // 2) & (cm >= DH // 2), 1.0, 0.0) \
        + jnp.where((r == c + DH // 2) & (cm < DH // 2), -1.0, 0.0)
    d = lax.broadcasted_iota(jnp.int32, (DH, HD), 0)
    cc = lax.broadcasted_iota(jnp.int32, (DH, HD), 1)
    tile = jnp.where(lax.rem(cc, DH) == d, 1.0, 0.0)
    return p.astype(F32), tile.astype(F32)


def _apply_rotary(t, remb):
    p, tile = _rot_mats()
    ct = jnp.dot(jnp.cos(remb), tile, preferred_element_type=F32)
    st = jnp.dot(jnp.sin(remb), tile, preferred_element_type=F32)
    rh = jnp.dot(t, p, preferred_element_type=F32)
    return t * ct + rh * st


def _proj_rot_kernel(g_ref, remb_ref, gamma_ref, w_ref, out_ref):
    tn = _layernorm(g_ref[0], gamma_ref[...])
    t = jnp.dot(tn, w_ref[...], preferred_element_type=F32)
    out_ref[0] = _apply_rotary(t, remb_ref[0])


def _proj_kv_kernel(g_ref, remb_ref, sc_ref, gamma_ref, wk_ref, wv_ref,
                    k_ref, v_ref):
    tn = _layernorm(g_ref[0], gamma_ref[...])
    t = jnp.dot(tn, wk_ref[...], preferred_element_type=F32)
    k_ref[0] = _apply_rotary(t, remb_ref[0])
    v = jnp.dot(tn, wv_ref[...], preferred_element_type=F32)
    v_ref[0] = v * sc_ref[0]


def _attn_kernel(q_ref, k_ref, v_ref, o_ref):
    outs = []
    for i in range(2):
        q = q_ref[0][:, i * DH:(i + 1) * DH]
        k = k_ref[0][:, i * DH:(i + 1) * DH]
        v = v_ref[0][:, i * DH:(i + 1) * DH]
        sim = lax.dot_general(q, k, (((1,), (1,)), ((), ())),
                              preferred_element_type=F32)
        sim = sim * (DH ** -0.5)
        m = jnp.max(sim, axis=1, keepdims=True)
        e = jnp.exp(sim - m)
        a = e / jnp.sum(e, axis=1, keepdims=True)
        outs.append(jnp.dot(a, v, preferred_element_type=F32))
    o_ref[0] = jnp.concatenate(outs, axis=1)


def _out_kernel(ao_ref, wo_ref, qs_ref, rank_ref, sel_ref, null_ref,
                out_ref, oh_scr, ol_scr):
    @pl.when(pl.program_id(1) == 0)
    def _project():
        o = jnp.dot(ao_ref[0], wo_ref[...], preferred_element_type=F32)
        o = o * qs_ref[0]
        hi, lo = _hilo(o)
        oh_scr[...] = hi
        ol_scr[...] = lo

    oh = _onehot(rank_ref[0, 0:1], sel_ref[0, 0:1], NQ)
    ohb = oh.astype(BF)
    dn_t = (((0,), (0,)), ((), ()))
    scat = lax.dot_general(ohb, oh_scr[...], dn_t,
                           preferred_element_type=F32) \
        + lax.dot_general(ohb, ol_scr[...], dn_t,
                          preferred_element_type=F32)
    selc = lax.dot_general(oh, jnp.ones((NQ, 1), F32), dn_t,
                           preferred_element_type=F32)
    out_ref[0] = scat + (1.0 - selc) * null_ref[...]


def _row_spec():
    return pl.BlockSpec((1, 1, N), lambda b: (b, 0, 0))


def _row_blk_spec():
    return pl.BlockSpec((1, 1, NBLK), lambda b, nb: (b, 0, nb))


def _full_spec(shape):
    return pl.BlockSpec(shape, lambda *_: (0,) * len(shape))


def _proj_rot(g, remb, g2, w, rows):
    return pl.pallas_call(
        _proj_rot_kernel,
        grid=(B,),
        in_specs=[
            pl.BlockSpec((1, rows, D), lambda b: (b, 0, 0)),
            pl.BlockSpec((1, rows, DH), lambda b: (b, 0, 0)),
            _full_spec((1, D)),
            _full_spec((D, HD)),
        ],
        out_specs=pl.BlockSpec((1, rows, HD), lambda b: (b, 0, 0)),
        out_shape=jax.ShapeDtypeStruct((B, rows, HD), F32),
    )(g, remb, g2, w)


@jax.jit
def kernel(x, rotary_emb, w_q_router, w_kv_router, ln_gamma, Wq, Wk, Wv, Wo,
           null_tokens):
    wq2 = w_q_router.reshape(1, D)
    wkv2 = w_kv_router.reshape(1, D)
    g2 = ln_gamma.reshape(1, D)
    null2 = null_tokens.reshape(1, D)

    row_f = jax.ShapeDtypeStruct((B, 1, N), F32)
    row_i = jax.ShapeDtypeStruct((B, 1, N), jnp.int32)
    qsel, qrank, qsig, kvsel, kvrank, kvsig = pl.pallas_call(
        _router_kernel,
        grid=(1,),
        in_specs=[
            _full_spec((B, N, D)),
            _full_spec((1, D)),
            _full_spec((1, D)),
        ],
        out_specs=[_full_spec((B, 1, N))] * 6,
        out_shape=[row_f, row_i, row_f, row_f, row_i, row_f],
    )(x, wq2, wkv2)

    qg, qremb, qs, kvg, kvremb, ks = pl.pallas_call(
        _gather_kernel,
        grid=(B, NB),
        in_specs=[
            pl.BlockSpec((1, NBLK, D), lambda b, nb: (b, nb, 0)),
            _row_blk_spec(), _row_blk_spec(), _row_blk_spec(),
            _row_blk_spec(), _row_blk_spec(), _row_blk_spec(),
            pl.BlockSpec((NBLK, DH), lambda b, nb: (nb, 0)),
        ],
        out_specs=[
            pl.BlockSpec((1, NQ, D), lambda b, nb: (b, 0, 0)),
            pl.BlockSpec((1, NQ, DH), lambda b, nb: (b, 0, 0)),
            pl.BlockSpec((1, NQ, 1), lambda b, nb: (b, 0, 0)),
            pl.BlockSpec((1, NKV, D), lambda b, nb: (b, 0, 0)),
            pl.BlockSpec((1, NKV, DH), lambda b, nb: (b, 0, 0)),
            pl.BlockSpec((1, NKV, 1), lambda b, nb: (b, 0, 0)),
        ],
        out_shape=[
            jax.ShapeDtypeStruct((B, NQ, D), F32),
            jax.ShapeDtypeStruct((B, NQ, DH), F32),
            jax.ShapeDtypeStruct((B, NQ, 1), F32),
            jax.ShapeDtypeStruct((B, NKV, D), F32),
            jax.ShapeDtypeStruct((B, NKV, DH), F32),
            jax.ShapeDtypeStruct((B, NKV, 1), F32),
        ],
    )(x, qrank, qsel, qsig, kvrank, kvsel, kvsig, rotary_emb)

    q = _proj_rot(qg, qremb, g2, Wq, NQ)
    RB = NKV // 2
    k, v = pl.pallas_call(
        _proj_kv_kernel,
        grid=(B, 2),
        in_specs=[
            pl.BlockSpec((1, RB, D), lambda b, r: (b, r, 0)),
            pl.BlockSpec((1, RB, DH), lambda b, r: (b, r, 0)),
            pl.BlockSpec((1, RB, 1), lambda b, r: (b, r, 0)),
            _full_spec((1, D)),
            _full_spec((D, HD)),
            _full_spec((D, HD)),
        ],
        out_specs=[
            pl.BlockSpec((1, RB, HD), lambda b, r: (b, r, 0)),
            pl.BlockSpec((1, RB, HD), lambda b, r: (b, r, 0)),
        ],
        out_shape=[
            jax.ShapeDtypeStruct((B, NKV, HD), F32),
            jax.ShapeDtypeStruct((B, NKV, HD), F32),
        ],
    )(kvg, kvremb, ks, g2, Wk, Wv)

    ao = pl.pallas_call(
        _attn_kernel,
        grid=(B, H // 2),
        in_specs=[
            pl.BlockSpec((1, NQ, 2 * DH), lambda b, h: (b, 0, h)),
            pl.BlockSpec((1, NKV, 2 * DH), lambda b, h: (b, 0, h)),
            pl.BlockSpec((1, NKV, 2 * DH), lambda b, h: (b, 0, h)),
        ],
        out_specs=pl.BlockSpec((1, NQ, 2 * DH), lambda b, h: (b, 0, h)),
        out_shape=jax.ShapeDtypeStruct((B, NQ, HD), F32),
    )(q, k, v)

    result = pl.pallas_call(
        _out_kernel,
        grid=(B, NB),
        in_specs=[
            pl.BlockSpec((1, NQ, HD), lambda b, nb: (b, 0, 0)),
            _full_spec((HD, D)),
            pl.BlockSpec((1, NQ, 1), lambda b, nb: (b, 0, 0)),
            _row_blk_spec(), _row_blk_spec(),
            _full_spec((1, D)),
        ],
        out_specs=pl.BlockSpec((1, NBLK, D), lambda b, nb: (b, nb, 0)),
        out_shape=jax.ShapeDtypeStruct((B, N, D), F32),
        scratch_shapes=[pltpu.VMEM((NQ, D), BF), pltpu.VMEM((NQ, D), BF)],
    )(ao, Wo, qs, qrank, qsel, null2)

    return result
